# baseline (speedup 1.0000x reference)
import functools

import jax
import jax.numpy as jnp
from jax import lax
from jax.experimental import pallas as pl
from jax.experimental.pallas import tpu as pltpu
from jax.experimental.pallas import tpu_sc as plsc

EMBD = 64
NSETS = 6
BATCH = 4096
RTOT = NSETS * BATCH
NWORK = 32
ROWS_PER_W = RTOT // NWORK
CHUNK = 128
NCHUNK = ROWS_PER_W // CHUNK
TILE = 256
NTILES = RTOT // TILE
TPB = BATCH // TILE


def _repack_body(top_ref, bot_ref, out_ref):
    out_ref[...] = jnp.concatenate([top_ref[...], bot_ref[...]], axis=1)


def _tc_repack(tbl, blk):
    V = tbl.shape[0]
    half_blocks = V // 2 // blk
    return pl.pallas_call(
        _repack_body,
        grid=(half_blocks,),
        in_specs=[
            pl.BlockSpec((blk, EMBD), lambda i: (i, 0)),
            pl.BlockSpec((blk, EMBD), lambda i, hb=half_blocks: (i + hb, 0)),
        ],
        out_specs=pl.BlockSpec((blk, 2 * EMBD), lambda i: (i, 0)),
        out_shape=jax.ShapeDtypeStruct((V // 2, 2 * EMBD), jnp.float32),
    )(tbl, tbl)


def _sc_gather(user2, poi2, uidx3, pidx3):
    mesh = plsc.VectorSubcoreMesh(core_axis_name="c", subcore_axis_name="s")
    row_ty = jax.ShapeDtypeStruct((RTOT, 2 * EMBD), jnp.float32)

    @functools.partial(
        pl.kernel,
        mesh=mesh,
        out_type=[row_ty, row_ty],
        scratch_types=[
            pltpu.VMEM((NCHUNK, CHUNK), jnp.int32),
            pltpu.VMEM((ROWS_PER_W, 2 * EMBD), jnp.float32),
            pltpu.SemaphoreType.DMA,
        ],
    )
    def gk(user_hbm, poi_hbm, uidx_hbm, pidx_hbm, out_u, out_p,
           idx_v, rows_v, sem):
        wid = lax.axis_index("s") * 2 + lax.axis_index("c")
        base = wid * ROWS_PER_W
        for table, idxh, out in ((user_hbm, uidx_hbm, out_u),
                                 (poi_hbm, pidx_hbm, out_p)):
            pltpu.sync_copy(idxh.at[wid], idx_v)
            copies = [
                pltpu.async_copy(table.at[idx_v.at[j]],
                                 rows_v.at[pl.ds(j * CHUNK, CHUNK)], sem)
                for j in range(NCHUNK)
            ]
            for c in copies:
                c.wait()
            pltpu.sync_copy(rows_v, out.at[pl.ds(base, ROWS_PER_W)])

    return gk(user2, poi2, uidx3, pidx3)


def _score_body(u_half, p_half, tidx_ref, uidx_ref, pidx_ref, uT_ref,
                pT_ref, projT_ref, timeT_ref, out_ref, neg_ref):
    i = pl.program_id(0)
    s = i // TPB
    nt = projT_ref.shape[1]
    tid = tidx_ref[0]
    iota_t = lax.broadcasted_iota(jnp.int32, (nt, TILE), 0)
    ohf = (tid == iota_t).astype(jnp.float32)
    HT = jnp.dot(projT_ref[...], ohf.astype(jnp.bfloat16),
                 preferred_element_type=jnp.float32)
    tsel = jnp.dot(timeT_ref[...], ohf,
                   preferred_element_type=jnp.float32)
    uhalf = uidx_ref[0] >= u_half
    phalf = pidx_ref[0] >= p_half
    urows = uT_ref[...].T
    prows = pT_ref[...].T
    u = jnp.where(uhalf, urows[EMBD:, :], urows[:EMBD, :])
    p = jnp.where(phalf, prows[EMBD:, :], prows[:EMBD, :])
    H3 = HT.reshape(EMBD, EMBD, TILE)
    v = jnp.sum(H3 * u[None, :, :], axis=1)
    score = jnp.sum((v + tsel) * p, axis=0)
    sp = jnp.log(1.0 + jnp.exp(-jnp.abs(score)))
    ls_pos = jnp.minimum(score, 0.0) - sp
    ls_neg = -jnp.maximum(score, 0.0) - sp
    out_ref[...] = (-ls_pos).reshape(1, 1, TILE)

    @pl.when(i == TPB)
    def _init():
        neg_ref[...] = jnp.zeros_like(neg_ref)

    @pl.when(i >= TPB)
    def _acc():
        rows = lax.broadcasted_iota(jnp.int32, (8, 128), 0)
        cols = lax.broadcasted_iota(jnp.int32, (8, 128), 1)
        mask = (rows == (s - 1)) & (cols == 0)
        neg_ref[...] += jnp.where(mask, -jnp.sum(ls_neg), 0.0)


def _tc_score(tidx3, uidx3, pidx3, uT, pT, projT_bf, timeT, u_half, p_half):
    return pl.pallas_call(
        functools.partial(_score_body, u_half, p_half),
        grid=(NTILES,),
        in_specs=[
            pl.BlockSpec((1, 1, TILE), lambda i: (i, 0, 0)),
            pl.BlockSpec((1, 1, TILE), lambda i: (i, 0, 0)),
            pl.BlockSpec((1, 1, TILE), lambda i: (i, 0, 0)),
            pl.BlockSpec((TILE, 2 * EMBD), lambda i: (i, 0)),
            pl.BlockSpec((TILE, 2 * EMBD), lambda i: (i, 0)),
            pl.BlockSpec(projT_bf.shape, lambda i: (0, 0)),
            pl.BlockSpec(timeT.shape, lambda i: (0, 0)),
        ],
        out_specs=[
            pl.BlockSpec((1, 1, TILE), lambda i: (i, 0, 0)),
            pl.BlockSpec((8, 128), lambda i: (0, 0)),
        ],
        out_shape=[
            jax.ShapeDtypeStruct((NTILES, 1, TILE), jnp.float32),
            jax.ShapeDtypeStruct((8, 128), jnp.float32),
        ],
    )(tidx3, uidx3, pidx3, uT, pT, projT_bf, timeT)


def kernel(pos_u, pos_t, pos_p, neg_u, neg_t, neg_p, NS, user_W, poi_W,
           time_W, proj_W):
    nneg = neg_u.shape[0]
    all_u = jnp.concatenate([pos_u[None, :], neg_u], 0).reshape(-1).astype(jnp.int32)
    all_t = jnp.concatenate([pos_t[None, :], neg_t], 0).reshape(-1).astype(jnp.int32)
    all_p = jnp.concatenate([pos_p[None, :], neg_p], 0).reshape(-1).astype(jnp.int32)
    u_half = user_W.shape[0] // 2
    p_half = poi_W.shape[0] // 2
    user2 = _tc_repack(user_W, 5000)
    poi2 = _tc_repack(poi_W, 5000)
    uidx_packed = jnp.where(all_u < u_half, all_u, all_u - u_half)
    pidx_packed = jnp.where(all_p < p_half, all_p, all_p - p_half)
    u_pairs, p_pairs = _sc_gather(
        user2, poi2,
        uidx_packed.reshape(NWORK, NCHUNK, CHUNK),
        pidx_packed.reshape(NWORK, NCHUNK, CHUNK))
    out_all, neg_out = _tc_score(
        all_t.reshape(NTILES, 1, TILE),
        all_u.reshape(NTILES, 1, TILE),
        all_p.reshape(NTILES, 1, TILE),
        u_pairs, p_pairs,
        proj_W.T.astype(jnp.bfloat16),
        time_W.T, u_half, p_half)
    pos = out_all.reshape(-1)[:BATCH]
    neg = neg_out[:nneg, 0]
    return (pos, neg)


# speedup vs baseline: 1.1128x; 1.1128x over previous
import functools

import jax
import jax.numpy as jnp
from jax import lax
from jax.experimental import pallas as pl
from jax.experimental.pallas import tpu as pltpu
from jax.experimental.pallas import tpu_sc as plsc

EMBD = 64
NSETS = 6
BATCH = 4096
RTOT = NSETS * BATCH
NWORK = 32
ROWS_PER_W = RTOT // NWORK
CHUNK = 128
NCHUNK = ROWS_PER_W // CHUNK
TILE = 256
NTILES = RTOT // TILE
TPB = BATCH // TILE


def _repack_body(inT_ref, out_ref):
    xt = inT_ref[...].T
    out_ref[...] = jnp.concatenate([xt, xt], axis=1)


def _tc_repack(tbl, blk_l):
    tblT = tbl.T
    V = tblT.shape[1]
    grid = (V + blk_l - 1) // blk_l
    return pl.pallas_call(
        _repack_body,
        grid=(grid,),
        in_specs=[pl.BlockSpec((EMBD, blk_l), lambda i: (0, i))],
        out_specs=pl.BlockSpec((blk_l, 2 * EMBD), lambda i: (i, 0)),
        out_shape=jax.ShapeDtypeStruct((V, 2 * EMBD), jnp.float32),
    )(tblT)


def _sc_gather(user2, poi2, uidx3, pidx3):
    mesh = plsc.VectorSubcoreMesh(core_axis_name="c", subcore_axis_name="s")
    row_ty = jax.ShapeDtypeStruct((RTOT, 2 * EMBD), jnp.float32)

    @functools.partial(
        pl.kernel,
        mesh=mesh,
        out_type=[row_ty, row_ty],
        scratch_types=[
            pltpu.VMEM((NCHUNK, CHUNK), jnp.int32),
            pltpu.VMEM((ROWS_PER_W, 2 * EMBD), jnp.float32),
            pltpu.SemaphoreType.DMA,
        ],
    )
    def gk(user_hbm, poi_hbm, uidx_hbm, pidx_hbm, out_u, out_p,
           idx_v, rows_v, sem):
        wid = lax.axis_index("s") * 2 + lax.axis_index("c")
        base = wid * ROWS_PER_W
        for table, idxh, out in ((user_hbm, uidx_hbm, out_u),
                                 (poi_hbm, pidx_hbm, out_p)):
            pltpu.sync_copy(idxh.at[wid], idx_v)
            copies = [
                pltpu.async_copy(table.at[idx_v.at[j]],
                                 rows_v.at[pl.ds(j * CHUNK, CHUNK)], sem)
                for j in range(NCHUNK)
            ]
            for c in copies:
                c.wait()
            pltpu.sync_copy(rows_v, out.at[pl.ds(base, ROWS_PER_W)])

    return gk(user2, poi2, uidx3, pidx3)


def _score_body(tidx_ref, uT_ref, pT_ref, projT_ref, timeT_ref,
                out_ref, neg_ref):
    i = pl.program_id(0)
    s = i // TPB
    nt = projT_ref.shape[1]
    tid = tidx_ref[0]
    iota_t = lax.broadcasted_iota(jnp.int32, (nt, TILE), 0)
    ohf = (tid == iota_t).astype(jnp.float32)
    HT = jnp.dot(projT_ref[...], ohf.astype(jnp.bfloat16),
                 preferred_element_type=jnp.float32)
    tsel = jnp.dot(timeT_ref[...], ohf,
                   preferred_element_type=jnp.float32)
    u = uT_ref[...].T[:EMBD, :]
    p = pT_ref[...].T[:EMBD, :]
    H3 = HT.reshape(EMBD, EMBD, TILE)
    v = jnp.sum(H3 * u[None, :, :], axis=1)
    score = jnp.sum((v + tsel) * p, axis=0)
    sp = jnp.log(1.0 + jnp.exp(-jnp.abs(score)))
    ls_pos = jnp.minimum(score, 0.0) - sp
    ls_neg = -jnp.maximum(score, 0.0) - sp
    out_ref[...] = (-ls_pos).reshape(1, 1, TILE)

    @pl.when(i == TPB)
    def _init():
        neg_ref[...] = jnp.zeros_like(neg_ref)

    @pl.when(i >= TPB)
    def _acc():
        rows = lax.broadcasted_iota(jnp.int32, (8, 128), 0)
        cols = lax.broadcasted_iota(jnp.int32, (8, 128), 1)
        mask = (rows == (s - 1)) & (cols == 0)
        neg_ref[...] += jnp.where(mask, -jnp.sum(ls_neg), 0.0)


def _tc_score(tidx3, uT, pT, projT_bf, timeT):
    return pl.pallas_call(
        _score_body,
        grid=(NTILES,),
        in_specs=[
            pl.BlockSpec((1, 1, TILE), lambda i: (i, 0, 0)),
            pl.BlockSpec((TILE, 2 * EMBD), lambda i: (i, 0)),
            pl.BlockSpec((TILE, 2 * EMBD), lambda i: (i, 0)),
            pl.BlockSpec(projT_bf.shape, lambda i: (0, 0)),
            pl.BlockSpec(timeT.shape, lambda i: (0, 0)),
        ],
        out_specs=[
            pl.BlockSpec((1, 1, TILE), lambda i: (i, 0, 0)),
            pl.BlockSpec((8, 128), lambda i: (0, 0)),
        ],
        out_shape=[
            jax.ShapeDtypeStruct((NTILES, 1, TILE), jnp.float32),
            jax.ShapeDtypeStruct((8, 128), jnp.float32),
        ],
    )(tidx3, uT, pT, projT_bf, timeT)


def kernel(pos_u, pos_t, pos_p, neg_u, neg_t, neg_p, NS, user_W, poi_W,
           time_W, proj_W):
    nneg = neg_u.shape[0]
    all_u = jnp.concatenate([pos_u[None, :], neg_u], 0).reshape(-1).astype(jnp.int32)
    all_t = jnp.concatenate([pos_t[None, :], neg_t], 0).reshape(-1).astype(jnp.int32)
    all_p = jnp.concatenate([pos_p[None, :], neg_p], 0).reshape(-1).astype(jnp.int32)
    user2 = _tc_repack(user_W, 2048)
    poi2 = _tc_repack(poi_W, 2048)
    u_pairs, p_pairs = _sc_gather(
        user2, poi2,
        all_u.reshape(NWORK, NCHUNK, CHUNK),
        all_p.reshape(NWORK, NCHUNK, CHUNK))
    out_all, neg_out = _tc_score(
        all_t.reshape(NTILES, 1, TILE),
        u_pairs, p_pairs,
        proj_W.T.astype(jnp.bfloat16),
        time_W.T)
    pos = out_all.reshape(-1)[:BATCH]
    neg = neg_out[:nneg, 0]
    return (pos, neg)


# speedup vs baseline: 1.2406x; 1.1148x over previous
import functools

import jax
import jax.numpy as jnp
from jax import lax
from jax.experimental import pallas as pl
from jax.experimental.pallas import tpu as pltpu
from jax.experimental.pallas import tpu_sc as plsc

EMBD = 64
NSETS = 6
BATCH = 4096
RTOT = NSETS * BATCH
NWORK = 32
ROWS_PER_W = RTOT // NWORK
CHUNK = 128
NCHUNK = ROWS_PER_W // CHUNK
TILE = 256
NTILES = RTOT // TILE
TPB = BATCH // TILE


RBLK = 2048


def _repack_body(inT_ref, out_ref):
    xt = inT_ref[...].T
    out_ref[...] = jnp.concatenate([xt[:RBLK // 2], xt[RBLK // 2:]], axis=1)


def _tc_repack(tbl, blk_l):
    tblT = tbl.T
    V = tblT.shape[1]
    grid = (V + blk_l - 1) // blk_l
    return pl.pallas_call(
        _repack_body,
        grid=(grid,),
        in_specs=[pl.BlockSpec((EMBD, blk_l), lambda i: (0, i))],
        out_specs=pl.BlockSpec((blk_l // 2, 2 * EMBD), lambda i: (i, 0)),
        out_shape=jax.ShapeDtypeStruct((grid * blk_l // 2, 2 * EMBD),
                                       jnp.float32),
    )(tblT)


def _sc_gather(user2, poi2, uidx3, pidx3):
    mesh = plsc.VectorSubcoreMesh(core_axis_name="c", subcore_axis_name="s")
    row_ty = jax.ShapeDtypeStruct((RTOT, 2 * EMBD), jnp.float32)

    @functools.partial(
        pl.kernel,
        mesh=mesh,
        out_type=[row_ty, row_ty],
        scratch_types=[
            pltpu.VMEM((NCHUNK, CHUNK), jnp.int32),
            pltpu.VMEM((ROWS_PER_W, 2 * EMBD), jnp.float32),
            pltpu.SemaphoreType.DMA,
        ],
    )
    def gk(user_hbm, poi_hbm, uidx_hbm, pidx_hbm, out_u, out_p,
           idx_v, rows_v, sem):
        wid = lax.axis_index("s") * 2 + lax.axis_index("c")
        base = wid * ROWS_PER_W
        for table, idxh, out in ((user_hbm, uidx_hbm, out_u),
                                 (poi_hbm, pidx_hbm, out_p)):
            pltpu.sync_copy(idxh.at[wid], idx_v)
            copies = [
                pltpu.async_copy(table.at[idx_v.at[j]],
                                 rows_v.at[pl.ds(j * CHUNK, CHUNK)], sem)
                for j in range(NCHUNK)
            ]
            for c in copies:
                c.wait()
            pltpu.sync_copy(rows_v, out.at[pl.ds(base, ROWS_PER_W)])

    return gk(user2, poi2, uidx3, pidx3)


def _score_body(tidx_ref, uidx_ref, pidx_ref, uT_ref, pT_ref, projT_ref,
                timeT_ref, out_ref, neg_ref):
    i = pl.program_id(0)
    s = i // TPB
    nt = projT_ref.shape[1]
    tid = tidx_ref[0]
    iota_t = lax.broadcasted_iota(jnp.int32, (nt, TILE), 0)
    ohf = (tid == iota_t).astype(jnp.float32)
    HT = jnp.dot(projT_ref[...], ohf.astype(jnp.bfloat16),
                 preferred_element_type=jnp.float32)
    tsel = jnp.dot(timeT_ref[...], ohf,
                   preferred_element_type=jnp.float32)
    uhalf = (uidx_ref[0] % RBLK) >= (RBLK // 2)
    phalf = (pidx_ref[0] % RBLK) >= (RBLK // 2)
    urows = uT_ref[...].T
    prows = pT_ref[...].T
    u = jnp.where(uhalf, urows[EMBD:, :], urows[:EMBD, :])
    p = jnp.where(phalf, prows[EMBD:, :], prows[:EMBD, :])
    H3 = HT.reshape(EMBD, EMBD, TILE)
    v = jnp.sum(H3 * u[None, :, :], axis=1)
    score = jnp.sum((v + tsel) * p, axis=0)
    sp = jnp.log(1.0 + jnp.exp(-jnp.abs(score)))
    ls_pos = jnp.minimum(score, 0.0) - sp
    ls_neg = -jnp.maximum(score, 0.0) - sp
    out_ref[...] = (-ls_pos).reshape(1, 1, TILE)

    @pl.when(i == TPB)
    def _init():
        neg_ref[...] = jnp.zeros_like(neg_ref)

    @pl.when(i >= TPB)
    def _acc():
        rows = lax.broadcasted_iota(jnp.int32, (8, 128), 0)
        cols = lax.broadcasted_iota(jnp.int32, (8, 128), 1)
        mask = (rows == (s - 1)) & (cols == 0)
        neg_ref[...] += jnp.where(mask, -jnp.sum(ls_neg), 0.0)


def _tc_score(tidx3, uidx3, pidx3, uT, pT, projT_bf, timeT):
    return pl.pallas_call(
        _score_body,
        grid=(NTILES,),
        in_specs=[
            pl.BlockSpec((1, 1, TILE), lambda i: (i, 0, 0)),
            pl.BlockSpec((1, 1, TILE), lambda i: (i, 0, 0)),
            pl.BlockSpec((1, 1, TILE), lambda i: (i, 0, 0)),
            pl.BlockSpec((TILE, 2 * EMBD), lambda i: (i, 0)),
            pl.BlockSpec((TILE, 2 * EMBD), lambda i: (i, 0)),
            pl.BlockSpec(projT_bf.shape, lambda i: (0, 0)),
            pl.BlockSpec(timeT.shape, lambda i: (0, 0)),
        ],
        out_specs=[
            pl.BlockSpec((1, 1, TILE), lambda i: (i, 0, 0)),
            pl.BlockSpec((8, 128), lambda i: (0, 0)),
        ],
        out_shape=[
            jax.ShapeDtypeStruct((NTILES, 1, TILE), jnp.float32),
            jax.ShapeDtypeStruct((8, 128), jnp.float32),
        ],
    )(tidx3, uidx3, pidx3, uT, pT, projT_bf, timeT)


def kernel(pos_u, pos_t, pos_p, neg_u, neg_t, neg_p, NS, user_W, poi_W,
           time_W, proj_W):
    nneg = neg_u.shape[0]
    all_u = jnp.concatenate([pos_u[None, :], neg_u], 0).reshape(-1).astype(jnp.int32)
    all_t = jnp.concatenate([pos_t[None, :], neg_t], 0).reshape(-1).astype(jnp.int32)
    all_p = jnp.concatenate([pos_p[None, :], neg_p], 0).reshape(-1).astype(jnp.int32)
    user2 = _tc_repack(user_W, RBLK)
    poi2 = _tc_repack(poi_W, RBLK)
    hb = RBLK // 2
    upacked = (all_u // RBLK) * hb + all_u % hb
    ppacked = (all_p // RBLK) * hb + all_p % hb
    u_pairs, p_pairs = _sc_gather(
        user2, poi2,
        upacked.reshape(NWORK, NCHUNK, CHUNK),
        ppacked.reshape(NWORK, NCHUNK, CHUNK))
    out_all, neg_out = _tc_score(
        all_t.reshape(NTILES, 1, TILE),
        all_u.reshape(NTILES, 1, TILE),
        all_p.reshape(NTILES, 1, TILE),
        u_pairs, p_pairs,
        proj_W.T.astype(jnp.bfloat16),
        time_W.T)
    pos = out_all.reshape(-1)[:BATCH]
    neg = neg_out[:nneg, 0]
    return (pos, neg)
